# Initial kernel scaffold; baseline (speedup 1.0000x reference)
import jax
import jax.numpy as jnp
from jax.experimental import pallas as pl


def kernel(extended_attention_mask, query, rel_embedding_output, graph_output, src, dst, Wq, bq, Wk, bk, Wv, bv, Wo, bo, ln1_g, ln1_b, W1, b1, W2, b2, ln2_g, ln2_b):
    raise NotImplementedError("write your pallas kernel here")


# speedup vs baseline: 1.1113x; 1.1113x over previous
import functools

import jax
import jax.numpy as jnp
import numpy as np
from jax import lax
from jax.experimental import pallas as pl
from jax.experimental.pallas import tpu as pltpu
from jax.experimental.pallas import tpu_sc as plsc

B = 16
LQ = 128
L = 1024
E = B * L
N = 10000
H = 128
NH = 8
DH = H // NH
EPS = 1e-12

NW = 32
SCH = 64
FCH = 32
CHG = 32
EPW = E // NW


@functools.cache
def _make_sc_gather():
    mesh = plsc.VectorSubcoreMesh(core_axis_name="c", subcore_axis_name="s")

    @functools.partial(
        pl.kernel,
        mesh=mesh,
        out_type=(
            jax.ShapeDtypeStruct((E, H), jnp.float32),
            jax.ShapeDtypeStruct((E, H), jnp.float32),
        ),
        scratch_types=[
            pltpu.VMEM((CHG,), jnp.int32),
            pltpu.VMEM((CHG, H), jnp.float32),
            pltpu.SemaphoreType.DMA,
        ],
    )
    def _sc_gather(graph_hbm, src_hbm, dst_hbm, srcf_out, valf_out,
                   idx_v, rows_v, sem):
        wid = lax.axis_index("s") * 2 + lax.axis_index("c")
        base = wid * EPW

        def step(j, _):
            off = pl.multiple_of(base + j * CHG, CHG)
            pltpu.sync_copy(src_hbm.at[pl.ds(off, CHG)], idx_v)
            pltpu.async_copy(graph_hbm.at[idx_v], rows_v, sem).wait()
            pltpu.sync_copy(rows_v, srcf_out.at[pl.ds(off, CHG)])
            pltpu.sync_copy(dst_hbm.at[pl.ds(off, CHG)], idx_v)
            pltpu.async_copy(graph_hbm.at[idx_v], rows_v, sem).wait()
            pltpu.sync_copy(rows_v, valf_out.at[pl.ds(off, CHG)])
            return 0

        lax.fori_loop(0, EPW // CHG, step, 0)

    return _sc_gather


def _dense_body(mask_ref, xq_ref, srcf_ref, rel_ref, val_ref,
                Wq_ref, bq_ref, Wk1_ref, Wk2_ref, bk_ref, Wv_ref, bv_ref,
                Wo_ref, bo_ref, ln1g_ref, ln1b_ref,
                W1_ref, b1_ref, W2_ref, b2_ref, ln2g_ref, ln2b_ref,
                qn_out, vn_out):
    f32 = jnp.float32

    def dot(a, b):
        return lax.dot_general(a, b, (((1,), (0,)), ((), ())),
                               preferred_element_type=f32)

    def lnorm(x, g, b):
        mu = jnp.mean(x, axis=-1, keepdims=True)
        var = jnp.mean((x - mu) ** 2, axis=-1, keepdims=True)
        return (x - mu) / jnp.sqrt(var + EPS) * g + b

    xq = xq_ref[0]
    srcf = srcf_ref[0]
    rel = rel_ref[0]
    val = val_ref[0]
    mask = mask_ref[0, 0]

    q = dot(xq, Wq_ref[...]) + bq_ref[...]
    k = dot(srcf, Wk1_ref[...]) + dot(rel, Wk2_ref[...]) + bk_ref[...]
    v = dot(val, Wv_ref[...]) + bv_ref[...]

    scale = f32(1.0 / np.sqrt(DH))
    qctx_parts = []
    vctx_parts = []
    for h in range(NH):
        qh = q[:, h * DH:(h + 1) * DH]
        kh = k[:, h * DH:(h + 1) * DH]
        vh = v[:, h * DH:(h + 1) * DH]
        s = lax.dot_general(qh, kh, (((1,), (1,)), ((), ())),
                            preferred_element_type=f32) * scale
        e = jnp.exp(s - jnp.max(s, axis=1, keepdims=True))
        p = e / jnp.sum(e, axis=1, keepdims=True)
        qctx_parts.append(dot(p, vh))
        s2 = s + mask[:, None]
        e2 = jnp.exp(s2 - jnp.max(s2, axis=0, keepdims=True))
        p2 = e2 / jnp.sum(e2, axis=0, keepdims=True)
        vctx_parts.append(
            lax.dot_general(p2, qh, (((0,), (0,)), ((), ())),
                            preferred_element_type=f32))

    qctx = jnp.concatenate(qctx_parts, axis=1)
    vctx = jnp.concatenate(vctx_parts, axis=1)

    hidden = dot(qctx, Wo_ref[...]) + bo_ref[...] + xq
    qn_out[0] = lnorm(hidden, ln1g_ref[...], ln1b_ref[...])

    theta = jax.nn.sigmoid(dot(vctx, W1_ref[...]) + b1_ref[...]
                           + dot(val, W2_ref[...]) + b2_ref[...])
    vn = theta * vctx + (1.0 - theta) * val
    vn_out[0] = lnorm(vn, ln2g_ref[...], ln2b_ref[...])


def _run_dense(mask, query, srcf, rel, val,
               Wq, bq, Wk1, Wk2, bk, Wv, bv, Wo, bo, ln1g, ln1b,
               W1, b1, W2, b2, ln2g, ln2b):
    def whole(shape):
        return pl.BlockSpec(shape, lambda b: (0,) * len(shape))

    def batched(shape):
        return pl.BlockSpec(shape, lambda b: (b,) + (0,) * (len(shape) - 1))

    in_specs = [
        batched((1, 1, LQ)),
        batched((1, LQ, H)),
        batched((1, L, H)),
        batched((1, L, H)),
        batched((1, L, H)),
        whole((H, H)), whole((1, H)),
        whole((H, H)), whole((H, H)), whole((1, H)),
        whole((H, H)), whole((1, H)),
        whole((H, H)), whole((1, H)),
        whole((1, H)), whole((1, H)),
        whole((H, H)), whole((1, H)),
        whole((H, H)), whole((1, H)),
        whole((1, H)), whole((1, H)),
    ]
    out_specs = (batched((1, LQ, H)), batched((1, L, H)))
    out_shape = (jax.ShapeDtypeStruct((B, LQ, H), jnp.float32),
                 jax.ShapeDtypeStruct((B, L, H), jnp.float32))
    return pl.pallas_call(
        _dense_body,
        grid=(B,),
        in_specs=in_specs,
        out_specs=out_specs,
        out_shape=out_shape,
    )(mask, query, srcf, rel, val, Wq, bq, Wk1, Wk2, bk, Wv, bv,
      Wo, bo, ln1g, ln1b, W1, b1, W2, b2, ln2g, ln2b)


@functools.cache
def _make_sc_scatter():
    mesh = plsc.VectorSubcoreMesh(core_axis_name="c", subcore_axis_name="s",
                                  num_cores=1)

    @functools.partial(
        pl.kernel,
        mesh=mesh,
        out_type=(jax.ShapeDtypeStruct((N, H), jnp.float32),
                  jax.ShapeDtypeStruct((N, H), jnp.float32)),
        scratch_types=[
            pltpu.VMEM_SHARED((N, H), jnp.float32),
            pltpu.VMEM((SCH,), jnp.int32),
            pltpu.VMEM((SCH, H), jnp.float32),
            pltpu.VMEM((SCH, H), jnp.float32),
            pltpu.VMEM((FCH, H), jnp.float32),
        ],
    )
    def _sc_scatter(vflat_hbm, dst_hbm, graph_hbm, out_hbm, sums_hbm,
                    acc_sh, idx_v, rows_v, ones_v, zeros_v):
        sid = lax.axis_index("s")

        def init_ones(i, _):
            for j in range(H // 16):
                ones_v[i, pl.ds(j * 16, 16)] = jnp.ones((16,), jnp.float32)
            return 0

        def init_zeros(i, _):
            for j in range(H // 16):
                zeros_v[i, pl.ds(j * 16, 16)] = jnp.zeros((16,), jnp.float32)
            return 0

        lax.fori_loop(0, SCH, init_ones, 0)
        lax.fori_loop(0, FCH, init_zeros, 0)

        NZ = N // FCH
        NTAIL = N - NZ * FCH

        def zero_acc(t, _):
            c = t * 16 + sid

            @pl.when(c < NZ)
            def _():
                zb = pl.multiple_of(c * FCH, FCH)
                pltpu.sync_copy(zeros_v, acc_sh.at[pl.ds(zb, FCH)])

            @pl.when(c == NZ)
            def _():
                pltpu.sync_copy(zeros_v.at[pl.ds(0, NTAIL)],
                                acc_sh.at[pl.ds(NZ * FCH, NTAIL)])
            return 0

        ebase = sid * (E // 16)

        def scat_vals(j, _):
            off = pl.multiple_of(ebase + j * SCH, SCH)
            pltpu.sync_copy(dst_hbm.at[pl.ds(off, SCH)], idx_v)
            pltpu.sync_copy(vflat_hbm.at[pl.ds(off, SCH)], rows_v)
            pltpu.sync_copy(rows_v, acc_sh.at[idx_v], add=True)
            return 0

        def scat_ones(j, _):
            off = pl.multiple_of(ebase + j * SCH, SCH)
            pltpu.sync_copy(dst_hbm.at[pl.ds(off, SCH)], idx_v)
            pltpu.sync_copy(ones_v, acc_sh.at[idx_v], add=True)
            return 0

        def drain_sums(t, _):
            c = t * 16 + sid

            @pl.when(c < NZ)
            def _():
                rb = pl.multiple_of(c * FCH, FCH)
                pltpu.sync_copy(acc_sh.at[pl.ds(rb, FCH)],
                                rows_v.at[pl.ds(0, FCH)])
                pltpu.sync_copy(rows_v.at[pl.ds(0, FCH)],
                                sums_hbm.at[pl.ds(rb, FCH)])

            @pl.when(c == NZ)
            def _():
                pltpu.sync_copy(acc_sh.at[pl.ds(NZ * FCH, NTAIL)],
                                rows_v.at[pl.ds(0, NTAIL)])
                pltpu.sync_copy(rows_v.at[pl.ds(0, NTAIL)],
                                sums_hbm.at[pl.ds(NZ * FCH, NTAIL)])
            return 0

        lax.fori_loop(0, (NZ + 16) // 16, zero_acc, 0)
        plsc.subcore_barrier()
        lax.fori_loop(0, E // 16 // SCH, scat_vals, 0)
        plsc.subcore_barrier()
        lax.fori_loop(0, (NZ + 16) // 16, drain_sums, 0)
        plsc.subcore_barrier()

        lax.fori_loop(0, (NZ + 16) // 16, zero_acc, 0)
        plsc.subcore_barrier()
        lax.fori_loop(0, E // 16 // SCH, scat_ones, 0)
        plsc.subcore_barrier()

        def fin_chunk(rb, clen):
            pltpu.sync_copy(sums_hbm.at[pl.ds(rb, clen)],
                            rows_v.at[pl.ds(0, clen)])
            pltpu.sync_copy(graph_hbm.at[pl.ds(rb, clen)],
                            rows_v.at[pl.ds(FCH, clen)])
            pltpu.sync_copy(acc_sh.at[pl.ds(rb, clen)],
                            ones_v.at[pl.ds(0, clen)])

            def fin_row(i, _):
                c = ones_v[i, pl.ds(0, 16)]
                recip = 1.0 / jnp.maximum(c, 1.0)
                pos = c > 0.0
                for j in range(H // 16):
                    csl = pl.ds(j * 16, 16)
                    mean = rows_v[i, csl] * recip
                    rows_v[i, csl] = jnp.where(pos, mean, rows_v[i + FCH, csl])
                return 0

            lax.fori_loop(0, clen, fin_row, 0)
            pltpu.sync_copy(rows_v.at[pl.ds(0, clen)],
                            out_hbm.at[pl.ds(rb, clen)])

        def fin_step(t, _):
            c = t * 16 + sid

            @pl.when(c < NZ)
            def _():
                fin_chunk(pl.multiple_of(c * FCH, FCH), FCH)

            @pl.when(c == NZ)
            def _():
                fin_chunk(NZ * FCH, NTAIL)
            return 0

        lax.fori_loop(0, (NZ + 16) // 16, fin_step, 0)

    return _sc_scatter


def kernel(extended_attention_mask, query, rel_embedding_output, graph_output,
           src, dst, Wq, bq, Wk, bk, Wv, bv, Wo, bo, ln1_g, ln1_b,
           W1, b1, W2, b2, ln2_g, ln2_b):
    src_feat, val_in = _make_sc_gather()(graph_output, src, dst)

    mask = extended_attention_mask.reshape(B, 1, LQ)
    rel = rel_embedding_output.reshape(B, L, H)
    srcf = src_feat.reshape(B, L, H)
    val = val_in.reshape(B, L, H)
    row = lambda x: x.reshape(1, H)

    query_new, value_new = _run_dense(
        mask, query, srcf, rel, val,
        Wq, row(bq), Wk[:H], Wk[H:], row(bk), Wv, row(bv),
        Wo, row(bo), row(ln1_g), row(ln1_b),
        W1, row(b1), W2, row(b2), row(ln2_g), row(ln2_b))

    graph_new, _sums = _make_sc_scatter()(value_new.reshape(E, H), dst,
                                          graph_output)
    return (graph_new, query_new)


# speedup vs baseline: 1.1831x; 1.0646x over previous
import functools

import jax
import jax.numpy as jnp
import numpy as np
from jax import lax
from jax.experimental import pallas as pl
from jax.experimental.pallas import tpu as pltpu
from jax.experimental.pallas import tpu_sc as plsc

B = 16
LQ = 128
L = 1024
E = B * L
N = 10000
H = 128
NH = 8
DH = H // NH
EPS = 1e-12

NW = 32
SCH = 64
FCH = 32
CHG = 32
EPW = E // NW


@functools.cache
def _make_sc_gather():
    mesh = plsc.VectorSubcoreMesh(core_axis_name="c", subcore_axis_name="s")

    @functools.partial(
        pl.kernel,
        mesh=mesh,
        out_type=(
            jax.ShapeDtypeStruct((E, H), jnp.float32),
            jax.ShapeDtypeStruct((E, H), jnp.float32),
        ),
        scratch_types=[
            pltpu.VMEM((CHG,), jnp.int32),
            pltpu.VMEM((CHG,), jnp.int32),
            pltpu.VMEM((CHG, H), jnp.float32),
            pltpu.VMEM((CHG, H), jnp.float32),
            pltpu.SemaphoreType.DMA,
            pltpu.SemaphoreType.DMA,
            pltpu.SemaphoreType.DMA,
            pltpu.SemaphoreType.DMA,
        ],
    )
    def _sc_gather(graph_hbm, src_hbm, dst_hbm, srcf_out, valf_out,
                   idx_a, idx_b, rows_a, rows_b, sem_a, sem_b,
                   sem_wa, sem_wb):
        wid = lax.axis_index("s") * 2 + lax.axis_index("c")
        base = wid * EPW
        nch = EPW // CHG

        def step(j, _):
            off = pl.multiple_of(base + j * CHG, CHG)
            pltpu.sync_copy(src_hbm.at[pl.ds(off, CHG)], idx_a)
            pltpu.sync_copy(dst_hbm.at[pl.ds(off, CHG)], idx_b)
            ga = pltpu.async_copy(graph_hbm.at[idx_a], rows_a, sem_a)
            gb = pltpu.async_copy(graph_hbm.at[idx_b], rows_b, sem_b)
            ga.wait()
            wa = pltpu.async_copy(rows_a, srcf_out.at[pl.ds(off, CHG)],
                                  sem_wa)
            gb.wait()
            wb = pltpu.async_copy(rows_b, valf_out.at[pl.ds(off, CHG)],
                                  sem_wb)
            wa.wait()
            wb.wait()
            return 0

        lax.fori_loop(0, nch, step, 0)

    return _sc_gather


def _dense_body(mask_ref, xq_ref, srcf_ref, rel_ref, val_ref,
                Wq_ref, bq_ref, Wk1_ref, Wk2_ref, bk_ref, Wv_ref, bv_ref,
                Wo_ref, bo_ref, ln1g_ref, ln1b_ref,
                W1_ref, b1_ref, W2_ref, b2_ref, ln2g_ref, ln2b_ref,
                qn_out, vn_out):
    f32 = jnp.float32

    def dot(a, b):
        return lax.dot_general(a, b, (((1,), (0,)), ((), ())),
                               preferred_element_type=f32)

    def lnorm(x, g, b):
        mu = jnp.mean(x, axis=-1, keepdims=True)
        var = jnp.mean((x - mu) ** 2, axis=-1, keepdims=True)
        return (x - mu) / jnp.sqrt(var + EPS) * g + b

    xq = xq_ref[0]
    srcf = srcf_ref[0]
    rel = rel_ref[0]
    val = val_ref[0]
    mask = mask_ref[0, 0]

    q = dot(xq, Wq_ref[...]) + bq_ref[...]
    k = dot(srcf, Wk1_ref[...]) + dot(rel, Wk2_ref[...]) + bk_ref[...]
    v = dot(val, Wv_ref[...]) + bv_ref[...]

    scale = f32(1.0 / np.sqrt(DH))
    qctx_parts = []
    vctx_parts = []
    for h in range(NH):
        qh = q[:, h * DH:(h + 1) * DH]
        kh = k[:, h * DH:(h + 1) * DH]
        vh = v[:, h * DH:(h + 1) * DH]
        s = lax.dot_general(qh, kh, (((1,), (1,)), ((), ())),
                            preferred_element_type=f32) * scale
        e = jnp.exp(s - jnp.max(s, axis=1, keepdims=True))
        p = e / jnp.sum(e, axis=1, keepdims=True)
        qctx_parts.append(dot(p, vh))
        s2 = s + mask[:, None]
        e2 = jnp.exp(s2 - jnp.max(s2, axis=0, keepdims=True))
        p2 = e2 / jnp.sum(e2, axis=0, keepdims=True)
        vctx_parts.append(
            lax.dot_general(p2, qh, (((0,), (0,)), ((), ())),
                            preferred_element_type=f32))

    qctx = jnp.concatenate(qctx_parts, axis=1)
    vctx = jnp.concatenate(vctx_parts, axis=1)

    hidden = dot(qctx, Wo_ref[...]) + bo_ref[...] + xq
    qn_out[0] = lnorm(hidden, ln1g_ref[...], ln1b_ref[...])

    theta = jax.nn.sigmoid(dot(vctx, W1_ref[...]) + b1_ref[...]
                           + dot(val, W2_ref[...]) + b2_ref[...])
    vn = theta * vctx + (1.0 - theta) * val
    vn_out[0] = lnorm(vn, ln2g_ref[...], ln2b_ref[...])


def _run_dense(mask, query, srcf, rel, val,
               Wq, bq, Wk1, Wk2, bk, Wv, bv, Wo, bo, ln1g, ln1b,
               W1, b1, W2, b2, ln2g, ln2b):
    def whole(shape):
        return pl.BlockSpec(shape, lambda b: (0,) * len(shape))

    def batched(shape):
        return pl.BlockSpec(shape, lambda b: (b,) + (0,) * (len(shape) - 1))

    in_specs = [
        batched((1, 1, LQ)),
        batched((1, LQ, H)),
        batched((1, L, H)),
        batched((1, L, H)),
        batched((1, L, H)),
        whole((H, H)), whole((1, H)),
        whole((H, H)), whole((H, H)), whole((1, H)),
        whole((H, H)), whole((1, H)),
        whole((H, H)), whole((1, H)),
        whole((1, H)), whole((1, H)),
        whole((H, H)), whole((1, H)),
        whole((H, H)), whole((1, H)),
        whole((1, H)), whole((1, H)),
    ]
    out_specs = (batched((1, LQ, H)), batched((1, L, H)))
    out_shape = (jax.ShapeDtypeStruct((B, LQ, H), jnp.float32),
                 jax.ShapeDtypeStruct((B, L, H), jnp.float32))
    return pl.pallas_call(
        _dense_body,
        grid=(B,),
        in_specs=in_specs,
        out_specs=out_specs,
        out_shape=out_shape,
    )(mask, query, srcf, rel, val, Wq, bq, Wk1, Wk2, bk, Wv, bv,
      Wo, bo, ln1g, ln1b, W1, b1, W2, b2, ln2g, ln2b)


@functools.cache
def _make_sc_scatter():
    mesh = plsc.VectorSubcoreMesh(core_axis_name="c", subcore_axis_name="s",
                                  num_cores=1)

    @functools.partial(
        pl.kernel,
        mesh=mesh,
        out_type=(jax.ShapeDtypeStruct((N, H), jnp.float32),
                  jax.ShapeDtypeStruct((N, H), jnp.float32)),
        scratch_types=[
            pltpu.VMEM_SHARED((N, H), jnp.float32),
            pltpu.VMEM((SCH,), jnp.int32),
            pltpu.VMEM((SCH,), jnp.int32),
            pltpu.VMEM((SCH, H), jnp.float32),
            pltpu.VMEM((SCH, H), jnp.float32),
            pltpu.VMEM((SCH, H), jnp.float32),
            pltpu.VMEM((FCH, H), jnp.float32),
            pltpu.SemaphoreType.DMA,
            pltpu.SemaphoreType.DMA,
        ],
    )
    def _sc_scatter(vflat_hbm, dst_hbm, graph_hbm, out_hbm, sums_hbm,
                    acc_sh, idx_v, idx_b, rows_v, rows_b, ones_v, zeros_v,
                    sem_sa, sem_sb):
        sid = lax.axis_index("s")

        def init_ones(i, _):
            for j in range(H // 16):
                ones_v[i, pl.ds(j * 16, 16)] = jnp.ones((16,), jnp.float32)
            return 0

        def init_zeros(i, _):
            for j in range(H // 16):
                zeros_v[i, pl.ds(j * 16, 16)] = jnp.zeros((16,), jnp.float32)
            return 0

        lax.fori_loop(0, SCH, init_ones, 0)
        lax.fori_loop(0, FCH, init_zeros, 0)

        NZ = N // FCH
        NTAIL = N - NZ * FCH

        def zero_acc(t, _):
            c = t * 16 + sid

            @pl.when(c < NZ)
            def _():
                zb = pl.multiple_of(c * FCH, FCH)
                pltpu.sync_copy(zeros_v, acc_sh.at[pl.ds(zb, FCH)])

            @pl.when(c == NZ)
            def _():
                pltpu.sync_copy(zeros_v.at[pl.ds(0, NTAIL)],
                                acc_sh.at[pl.ds(NZ * FCH, NTAIL)])
            return 0

        ebase = sid * (E // 16)

        def scat_vals(t, _):
            off_a = pl.multiple_of(ebase + (2 * t) * SCH, SCH)
            off_b = pl.multiple_of(ebase + (2 * t + 1) * SCH, SCH)
            pltpu.sync_copy(dst_hbm.at[pl.ds(off_a, SCH)], idx_v)
            pltpu.sync_copy(vflat_hbm.at[pl.ds(off_a, SCH)], rows_v)
            sa = pltpu.async_copy(rows_v, acc_sh.at[idx_v], sem_sa, add=True)
            pltpu.sync_copy(dst_hbm.at[pl.ds(off_b, SCH)], idx_b)
            pltpu.sync_copy(vflat_hbm.at[pl.ds(off_b, SCH)], rows_b)
            sb = pltpu.async_copy(rows_b, acc_sh.at[idx_b], sem_sb, add=True)
            sa.wait()
            sb.wait()
            return 0

        def scat_ones(t, _):
            off_a = pl.multiple_of(ebase + (2 * t) * SCH, SCH)
            off_b = pl.multiple_of(ebase + (2 * t + 1) * SCH, SCH)
            pltpu.sync_copy(dst_hbm.at[pl.ds(off_a, SCH)], idx_v)
            sa = pltpu.async_copy(ones_v, acc_sh.at[idx_v], sem_sa, add=True)
            pltpu.sync_copy(dst_hbm.at[pl.ds(off_b, SCH)], idx_b)
            sb = pltpu.async_copy(ones_v, acc_sh.at[idx_b], sem_sb, add=True)
            sa.wait()
            sb.wait()
            return 0

        def drain_sums(t, _):
            c = t * 16 + sid

            @pl.when(c < NZ)
            def _():
                rb = pl.multiple_of(c * FCH, FCH)
                pltpu.sync_copy(acc_sh.at[pl.ds(rb, FCH)],
                                rows_v.at[pl.ds(0, FCH)])
                pltpu.sync_copy(rows_v.at[pl.ds(0, FCH)],
                                sums_hbm.at[pl.ds(rb, FCH)])

            @pl.when(c == NZ)
            def _():
                pltpu.sync_copy(acc_sh.at[pl.ds(NZ * FCH, NTAIL)],
                                rows_v.at[pl.ds(0, NTAIL)])
                pltpu.sync_copy(rows_v.at[pl.ds(0, NTAIL)],
                                sums_hbm.at[pl.ds(NZ * FCH, NTAIL)])
            return 0

        lax.fori_loop(0, (NZ + 16) // 16, zero_acc, 0)
        plsc.subcore_barrier()
        lax.fori_loop(0, E // 16 // SCH // 2, scat_vals, 0)
        plsc.subcore_barrier()
        lax.fori_loop(0, (NZ + 16) // 16, drain_sums, 0)
        plsc.subcore_barrier()

        lax.fori_loop(0, (NZ + 16) // 16, zero_acc, 0)
        plsc.subcore_barrier()
        lax.fori_loop(0, E // 16 // SCH // 2, scat_ones, 0)
        plsc.subcore_barrier()

        def fin_chunk(rb, clen):
            pltpu.sync_copy(sums_hbm.at[pl.ds(rb, clen)],
                            rows_v.at[pl.ds(0, clen)])
            pltpu.sync_copy(graph_hbm.at[pl.ds(rb, clen)],
                            rows_v.at[pl.ds(FCH, clen)])
            pltpu.sync_copy(acc_sh.at[pl.ds(rb, clen)],
                            ones_v.at[pl.ds(0, clen)])

            def fin_row(i, _):
                c = ones_v[i, pl.ds(0, 16)]
                recip = 1.0 / jnp.maximum(c, 1.0)
                pos = c > 0.0
                for j in range(H // 16):
                    csl = pl.ds(j * 16, 16)
                    mean = rows_v[i, csl] * recip
                    rows_v[i, csl] = jnp.where(pos, mean, rows_v[i + FCH, csl])
                return 0

            lax.fori_loop(0, clen, fin_row, 0)
            pltpu.sync_copy(rows_v.at[pl.ds(0, clen)],
                            out_hbm.at[pl.ds(rb, clen)])

        def fin_step(t, _):
            c = t * 16 + sid

            @pl.when(c < NZ)
            def _():
                fin_chunk(pl.multiple_of(c * FCH, FCH), FCH)

            @pl.when(c == NZ)
            def _():
                fin_chunk(NZ * FCH, NTAIL)
            return 0

        lax.fori_loop(0, (NZ + 16) // 16, fin_step, 0)

    return _sc_scatter


def kernel(extended_attention_mask, query, rel_embedding_output, graph_output,
           src, dst, Wq, bq, Wk, bk, Wv, bv, Wo, bo, ln1_g, ln1_b,
           W1, b1, W2, b2, ln2_g, ln2_b):
    src_feat, val_in = _make_sc_gather()(graph_output, src, dst)

    mask = extended_attention_mask.reshape(B, 1, LQ)
    rel = rel_embedding_output.reshape(B, L, H)
    srcf = src_feat.reshape(B, L, H)
    val = val_in.reshape(B, L, H)
    row = lambda x: x.reshape(1, H)

    query_new, value_new = _run_dense(
        mask, query, srcf, rel, val,
        Wq, row(bq), Wk[:H], Wk[H:], row(bk), Wv, row(bv),
        Wo, row(bo), row(ln1_g), row(ln1_b),
        W1, row(b1), W2, row(b2), row(ln2_g), row(ln2_b))

    graph_new, _sums = _make_sc_scatter()(value_new.reshape(E, H), dst,
                                          graph_output)
    return (graph_new, query_new)
